# baseline (speedup 1.0000x reference)
import math

import jax
import jax.numpy as jnp
from jax.experimental import pallas as pl

BLK = 1000


def _body(h_ref, dt_ref, f_ref, tab_ref, w1_ref, b1_ref, w2_ref, b2_ref, o_ref):
    blk = h_ref.shape[0]
    hcol = h_ref[...]
    kiota = jax.lax.broadcasted_iota(jnp.int32, (blk, 128), 1)
    oh = (hcol == kiota).astype(jnp.float32)
    arow = jnp.dot(oh, tab_ref[...], preferred_element_type=jnp.float32)

    cat = jnp.concatenate([arow, f_ref[...]], axis=1)
    hf = jax.lax.dot_general(cat, w1_ref[...], (((1,), (1,)), ((), ())),
                             preferred_element_type=jnp.float32) + b1_ref[...]

    dt = dt_ref[...]
    jiota = jax.lax.broadcasted_iota(jnp.int32, (1, 64), 1).astype(jnp.float32)
    freqs = jnp.exp(jiota * (-math.log(10000.0) / 63.0))
    arg = dt * freqs
    temb = jnp.concatenate([jnp.sin(arg), jnp.cos(arg)], axis=1)

    cat2 = jnp.concatenate([hf, temb], axis=1)
    row = jax.lax.dot_general(cat2, w2_ref[...], (((1,), (1,)), ((), ())),
                              preferred_element_type=jnp.float32) + b2_ref[...]

    t = o_ref.shape[1] // row.shape[1]
    idx = jax.lax.broadcasted_iota(jnp.int32, (blk, 128 * t), 1) // t
    lo = jnp.take_along_axis(row[:, :128], idx, axis=1)
    hi = jnp.take_along_axis(row[:, 128:], idx, axis=1)
    o_ref[...] = jnp.concatenate([lo, hi], axis=1)


def kernel(diffusion_t, x, h, f, edge_index, edge_attr, batch, atom_emb,
           emb_lin_W, emb_lin_b, edge_emb_table, input_lin_W, input_lin_b,
           cond_emb_table):
    bn = x.shape[0]
    t = x.shape[-1]
    node_dim = atom_emb.shape[1]

    h_col = h.astype(jnp.int32).reshape(bn, 1)
    dt_col = diffusion_t.astype(jnp.float32).reshape(bn, 1)
    tab = jnp.zeros((128, node_dim), jnp.float32).at[:atom_emb.shape[0]].set(atom_emb)
    b1 = emb_lin_b.reshape(1, -1)
    b2 = input_lin_b.reshape(1, -1)

    grid = bn // BLK
    out2d = pl.pallas_call(
        _body,
        grid=(grid,),
        in_specs=[
            pl.BlockSpec((BLK, 1), lambda i: (i, 0)),
            pl.BlockSpec((BLK, 1), lambda i: (i, 0)),
            pl.BlockSpec((BLK, f.shape[1]), lambda i: (i, 0)),
            pl.BlockSpec(tab.shape, lambda i: (0, 0)),
            pl.BlockSpec(emb_lin_W.shape, lambda i: (0, 0)),
            pl.BlockSpec((1, b1.shape[1]), lambda i: (0, 0)),
            pl.BlockSpec(input_lin_W.shape, lambda i: (0, 0)),
            pl.BlockSpec((1, b2.shape[1]), lambda i: (0, 0)),
        ],
        out_specs=pl.BlockSpec((BLK, node_dim * t), lambda i: (i, 0)),
        out_shape=jax.ShapeDtypeStruct((bn, node_dim * t), jnp.float32),
    )(h_col, dt_col, f, tab, emb_lin_W, b1, input_lin_W, b2)
    return out2d.reshape(bn, node_dim, t)


# speedup vs baseline: 1.0408x; 1.0408x over previous
import math

import jax
import jax.numpy as jnp
from jax.experimental import pallas as pl

BLK = 1000


def _body(h_ref, dt_ref, f_ref, tab_ref, w1_ref, b1_ref, w2_ref, b2_ref, o_ref):
    blk = f_ref.shape[0]
    nvocab = tab_ref.shape[0]

    hcol = jnp.transpose(h_ref[0], (1, 0))
    kiota = jax.lax.broadcasted_iota(jnp.int32, (blk, nvocab), 1)
    oh = (hcol == kiota).astype(jnp.float32)
    arow = jnp.dot(oh, tab_ref[...], preferred_element_type=jnp.float32)

    cat = jnp.concatenate([arow, f_ref[...]], axis=1)
    hf = jax.lax.dot_general(cat, w1_ref[...], (((1,), (1,)), ((), ())),
                             preferred_element_type=jnp.float32) + b1_ref[...]

    dt = jnp.transpose(dt_ref[0], (1, 0)).astype(jnp.float32)
    jiota = jax.lax.broadcasted_iota(jnp.int32, (1, 64), 1).astype(jnp.float32)
    freqs = jnp.exp(jiota * (-math.log(10000.0) / 63.0))
    arg = dt * freqs
    temb = jnp.concatenate([jnp.sin(arg), jnp.cos(arg)], axis=1)

    cat2 = jnp.concatenate([hf, temb], axis=1)
    row = jax.lax.dot_general(cat2, w2_ref[...], (((1,), (1,)), ((), ())),
                              preferred_element_type=jnp.float32) + b2_ref[...]

    t = o_ref.shape[1] // row.shape[1]
    idx = jax.lax.broadcasted_iota(jnp.int32, (blk, 128 * t), 1) // t
    lo = jnp.take_along_axis(row[:, :128], idx, axis=1)
    hi = jnp.take_along_axis(row[:, 128:], idx, axis=1)
    o_ref[...] = jnp.concatenate([lo, hi], axis=1)


def kernel(diffusion_t, x, h, f, edge_index, edge_attr, batch, atom_emb,
           emb_lin_W, emb_lin_b, edge_emb_table, input_lin_W, input_lin_b,
           cond_emb_table):
    bn = x.shape[0]
    t = x.shape[-1]
    node_dim = atom_emb.shape[1]
    grid = bn // BLK

    h3 = h.astype(jnp.int32).reshape(grid, 1, BLK)
    dt3 = diffusion_t.astype(jnp.int32).reshape(grid, 1, BLK)
    b1 = emb_lin_b.reshape(1, -1)
    b2 = input_lin_b.reshape(1, -1)

    out2d = pl.pallas_call(
        _body,
        grid=(grid,),
        in_specs=[
            pl.BlockSpec((1, 1, BLK), lambda i: (i, 0, 0)),
            pl.BlockSpec((1, 1, BLK), lambda i: (i, 0, 0)),
            pl.BlockSpec((BLK, f.shape[1]), lambda i: (i, 0)),
            pl.BlockSpec(atom_emb.shape, lambda i: (0, 0)),
            pl.BlockSpec(emb_lin_W.shape, lambda i: (0, 0)),
            pl.BlockSpec((1, b1.shape[1]), lambda i: (0, 0)),
            pl.BlockSpec(input_lin_W.shape, lambda i: (0, 0)),
            pl.BlockSpec((1, b2.shape[1]), lambda i: (0, 0)),
        ],
        out_specs=pl.BlockSpec((BLK, node_dim * t), lambda i: (i, 0)),
        out_shape=jax.ShapeDtypeStruct((bn, node_dim * t), jnp.float32),
    )(h3, dt3, f, atom_emb, emb_lin_W, b1, input_lin_W, b2)
    return out2d.reshape(bn, node_dim, t)


# speedup vs baseline: 2.2094x; 2.1228x over previous
import math

import jax
import jax.numpy as jnp
from jax.experimental import pallas as pl

BLK = 1000


def _body(h_ref, dt_ref, f_ref, tab_ref, w1_ref, b1_ref, w2_ref, b2_ref, o_ref):
    blk = f_ref.shape[0]
    nvocab = tab_ref.shape[0]

    hcol = jnp.transpose(h_ref[0], (1, 0))
    kiota = jax.lax.broadcasted_iota(jnp.int32, (blk, nvocab), 1)
    oh = (hcol == kiota).astype(jnp.float32)
    arow = jnp.dot(oh, tab_ref[...], preferred_element_type=jnp.float32)

    cat = jnp.concatenate([arow, f_ref[...]], axis=1)
    hf = jax.lax.dot_general(cat, w1_ref[...], (((1,), (1,)), ((), ())),
                             preferred_element_type=jnp.float32) + b1_ref[...]

    dt = jnp.transpose(dt_ref[0], (1, 0)).astype(jnp.float32)
    jiota = jax.lax.broadcasted_iota(jnp.int32, (1, 64), 1).astype(jnp.float32)
    freqs = jnp.exp(jiota * (-math.log(10000.0) / 63.0))
    arg = dt * freqs
    temb = jnp.concatenate([jnp.sin(arg), jnp.cos(arg)], axis=1)

    cat2 = jnp.concatenate([hf, temb], axis=1)
    row = jax.lax.dot_general(cat2, w2_ref[...], (((1,), (1,)), ((), ())),
                              preferred_element_type=jnp.float32) + b2_ref[...]

    t = o_ref.shape[1] // row.shape[1]
    o_ref[...] = jnp.concatenate([row] * t, axis=1)


def kernel(diffusion_t, x, h, f, edge_index, edge_attr, batch, atom_emb,
           emb_lin_W, emb_lin_b, edge_emb_table, input_lin_W, input_lin_b,
           cond_emb_table):
    bn = x.shape[0]
    t = x.shape[-1]
    node_dim = atom_emb.shape[1]
    grid = bn // BLK

    h3 = h.astype(jnp.int32).reshape(grid, 1, BLK)
    dt3 = diffusion_t.astype(jnp.int32).reshape(grid, 1, BLK)
    b1 = emb_lin_b.reshape(1, -1)
    b2 = input_lin_b.reshape(1, -1)

    out2d = pl.pallas_call(
        _body,
        grid=(grid,),
        in_specs=[
            pl.BlockSpec((1, 1, BLK), lambda i: (i, 0, 0)),
            pl.BlockSpec((1, 1, BLK), lambda i: (i, 0, 0)),
            pl.BlockSpec((BLK, f.shape[1]), lambda i: (i, 0)),
            pl.BlockSpec(atom_emb.shape, lambda i: (0, 0)),
            pl.BlockSpec(emb_lin_W.shape, lambda i: (0, 0)),
            pl.BlockSpec((1, b1.shape[1]), lambda i: (0, 0)),
            pl.BlockSpec(input_lin_W.shape, lambda i: (0, 0)),
            pl.BlockSpec((1, b2.shape[1]), lambda i: (0, 0)),
        ],
        out_specs=pl.BlockSpec((BLK, node_dim * t), lambda i: (i, 0)),
        out_shape=jax.ShapeDtypeStruct((bn, node_dim * t), jnp.float32),
    )(h3, dt3, f, atom_emb, emb_lin_W, b1, input_lin_W, b2)
    return out2d.reshape(bn, t, node_dim).transpose(0, 2, 1)


# speedup vs baseline: 2.2171x; 1.0035x over previous
import math

import jax
import jax.numpy as jnp
from jax.experimental import pallas as pl

BLK = 1000


def _body(h_ref, dt_ref, f_ref, tab_ref, w1_ref, b1_ref, w2_ref, b2_ref, o_ref):
    blk = f_ref.shape[0]
    nvocab = tab_ref.shape[0]

    hrow = h_ref[0]
    kiota = jax.lax.broadcasted_iota(jnp.int32, (nvocab, blk), 0)
    oh2 = (hrow == kiota).astype(jnp.float32)
    arow = jax.lax.dot_general(oh2, tab_ref[...], (((0,), (0,)), ((), ())),
                               preferred_element_type=jnp.float32)

    cat = jnp.concatenate([arow, f_ref[...]], axis=1)
    hf = jax.lax.dot_general(cat, w1_ref[...], (((1,), (1,)), ((), ())),
                             preferred_element_type=jnp.float32) + b1_ref[...]

    dtrow = dt_ref[0].astype(jnp.float32)
    jiota = jax.lax.broadcasted_iota(jnp.int32, (1, 64), 1).astype(jnp.float32)
    freqs = jnp.exp(jiota * (-math.log(10000.0) / 63.0))
    arg = jax.lax.dot_general(dtrow, freqs, (((0,), (0,)), ((), ())),
                              preferred_element_type=jnp.float32)
    temb = jnp.concatenate([jnp.sin(arg), jnp.cos(arg)], axis=1)

    cat2 = jnp.concatenate([hf, temb], axis=1)
    row = jax.lax.dot_general(cat2, w2_ref[...], (((1,), (1,)), ((), ())),
                              preferred_element_type=jnp.float32) + b2_ref[...]

    t = o_ref.shape[1] // row.shape[1]
    o_ref[...] = jnp.concatenate([row] * t, axis=1)


def kernel(diffusion_t, x, h, f, edge_index, edge_attr, batch, atom_emb,
           emb_lin_W, emb_lin_b, edge_emb_table, input_lin_W, input_lin_b,
           cond_emb_table):
    bn = x.shape[0]
    t = x.shape[-1]
    node_dim = atom_emb.shape[1]
    grid = bn // BLK

    h3 = h.astype(jnp.int32).reshape(grid, 1, BLK)
    dt3 = diffusion_t.astype(jnp.int32).reshape(grid, 1, BLK)
    b1 = emb_lin_b.reshape(1, -1)
    b2 = input_lin_b.reshape(1, -1)

    out2d = pl.pallas_call(
        _body,
        grid=(grid,),
        in_specs=[
            pl.BlockSpec((1, 1, BLK), lambda i: (i, 0, 0)),
            pl.BlockSpec((1, 1, BLK), lambda i: (i, 0, 0)),
            pl.BlockSpec((BLK, f.shape[1]), lambda i: (i, 0)),
            pl.BlockSpec(atom_emb.shape, lambda i: (0, 0)),
            pl.BlockSpec(emb_lin_W.shape, lambda i: (0, 0)),
            pl.BlockSpec((1, b1.shape[1]), lambda i: (0, 0)),
            pl.BlockSpec(input_lin_W.shape, lambda i: (0, 0)),
            pl.BlockSpec((1, b2.shape[1]), lambda i: (0, 0)),
        ],
        out_specs=pl.BlockSpec((BLK, node_dim * t), lambda i: (i, 0)),
        out_shape=jax.ShapeDtypeStruct((bn, node_dim * t), jnp.float32),
    )(h3, dt3, f, atom_emb, emb_lin_W, b1, input_lin_W, b2)
    return out2d.reshape(bn, t, node_dim).transpose(0, 2, 1)
